# Initial kernel scaffold; baseline (speedup 1.0000x reference)
import jax
import jax.numpy as jnp
from jax.experimental import pallas as pl


def kernel(x, coords, s, pixel_offset_normal, pixel_offset_indices):
    raise NotImplementedError("write your pallas kernel here")


# speedup vs baseline: 1.7456x; 1.7456x over previous
import functools

import jax
import jax.numpy as jnp
from jax import lax
from jax.experimental import pallas as pl
from jax.experimental.pallas import tpu as pltpu
from jax.experimental.pallas import tpu_sc as plsc

NH = 5
N_RES = 90
NBIN = N_RES // NH
NSLOT = NH * NH
NW = 32
PPW = 128


def _prep_body(cx_ref, cy_ref, s_ref, pon_ref, poi_ref, idx_ref, w_ref,
               *, nx, ny, npix):
    px = cx_ref[...] * (nx - 1)
    py = cy_ref[...] * (ny - 1)
    rpx = jnp.round(px)
    rpy = jnp.round(py)
    s = s_ref[0]
    cexp = -0.5 / (s * s)

    wx, wy = [], []
    for i in range(NH):
        ax = None
        ay = None
        for k in range(NBIN):
            o = pon_ref[i * NBIN + k]
            vx = jnp.clip(rpx - o, 0.0, float(nx))
            vy = jnp.clip(rpy - o, 0.0, float(nx))
            tx = jnp.exp(cexp * (vx - px) ** 2)
            ty = jnp.exp(cexp * (vy - py) ** 2)
            ax = tx if ax is None else ax + tx
            ay = ty if ay is None else ay + ty
        wx.append(ax)
        wy.append(ay)
    zx = wx[0] + wx[1] + wx[2] + wx[3] + wx[4]
    zy = wy[0] + wy[1] + wy[2] + wy[3] + wy[4]
    inv_z = 1.0 / (zx * zy)
    wxn = [w * inv_z for w in wx]

    rx, ry = [], []
    for i in range(NH):
        o = poi_ref[i]
        rx.append(jnp.clip(jnp.round(rpx - o), 0, nx - 1).astype(jnp.int32))
        ry.append(jnp.clip(jnp.round(rpy - o), 0, nx - 1).astype(jnp.int32))
    base = (lax.broadcasted_iota(jnp.int32, (NW, PPW), 0) // 8) * npix
    for i in range(NH):
        row = base + rx[i] * ny
        for j in range(NH):
            sl = i * NH + j
            idx_ref[:, sl, :] = row + ry[j]
            w_ref[:, sl, :] = wxn[i] * wy[j]


def _make_prep(nx, ny, interpret=False):
    body = functools.partial(_prep_body, nx=nx, ny=ny, npix=nx * ny)
    return pl.pallas_call(
        body,
        out_shape=[
            jax.ShapeDtypeStruct((NW, NSLOT, PPW), jnp.int32),
            jax.ShapeDtypeStruct((NW, NSLOT, PPW), jnp.float32),
        ],
        in_specs=[
            pl.BlockSpec(memory_space=pltpu.VMEM),
            pl.BlockSpec(memory_space=pltpu.VMEM),
            pl.BlockSpec(memory_space=pltpu.SMEM),
            pl.BlockSpec(memory_space=pltpu.SMEM),
            pl.BlockSpec(memory_space=pltpu.SMEM),
        ],
        out_specs=[
            pl.BlockSpec(memory_space=pltpu.VMEM),
            pl.BlockSpec(memory_space=pltpu.VMEM),
        ],
        interpret=interpret,
    )


def _sc_body(xt_hbm, idx_hbm, w_hbm, out_hbm,
             idx_v, w_v, rows_a, rows_b, acc_v, sem_a, sem_b, *, nch):
    wid = lax.axis_index("s") * 2 + lax.axis_index("c")
    pltpu.sync_copy(idx_hbm.at[wid], idx_v)
    pltpu.sync_copy(w_hbm.at[wid], w_v)
    nvec = nch // 16
    zeros = jnp.zeros((16,), jnp.float32)

    pltpu.make_async_copy(xt_hbm.at[idx_v.at[0]], rows_a, sem_a).start()

    def zbody(k, _):
        for cv in range(nvec):
            acc_v[k, pl.ds(cv * 16, 16)] = zeros
        return 0

    lax.fori_loop(0, PPW, zbody, 0)

    def slot_body(sl, _):
        def step(cur, nxt, csem, nsem):
            @pl.when(sl + 1 < NSLOT)
            def _start_next():
                pltpu.make_async_copy(
                    xt_hbm.at[idx_v.at[sl + 1]], nxt, nsem).start()

            pltpu.make_async_copy(
                xt_hbm.at[idx_v.at[sl]], cur, csem).wait()

            def kbody(kg, _2):
                wvec = w_v[sl, pl.ds(kg * 16, 16)]
                for k16 in range(16):
                    wk = jnp.broadcast_to(wvec[k16], (16,))
                    k = kg * 16 + k16
                    for cv in range(nvec):
                        r = cur[k, pl.ds(cv * 16, 16)]
                        acc_v[k, pl.ds(cv * 16, 16)] = (
                            acc_v[k, pl.ds(cv * 16, 16)] + wk * r)
                return 0

            lax.fori_loop(0, PPW // 16, kbody, 0)

        @pl.when(sl % 2 == 0)
        def _even():
            step(rows_a, rows_b, sem_a, sem_b)

        @pl.when(sl % 2 == 1)
        def _odd():
            step(rows_b, rows_a, sem_b, sem_a)

        return 0

    lax.fori_loop(0, NSLOT, slot_body, 0)
    pltpu.sync_copy(acc_v, out_hbm.at[pl.ds(wid * PPW, PPW)])


def _make_sc(npts, nch):
    mesh = plsc.VectorSubcoreMesh(core_axis_name="c", subcore_axis_name="s")
    return functools.partial(
        pl.kernel,
        mesh=mesh,
        compiler_params=pltpu.CompilerParams(use_tc_tiling_on_sc=False),
        out_type=jax.ShapeDtypeStruct((npts, nch), jnp.float32),
        scratch_types=[
            pltpu.VMEM((NSLOT, PPW), jnp.int32),
            pltpu.VMEM((NSLOT, PPW), jnp.float32),
            pltpu.VMEM((PPW, nch), jnp.float32),
            pltpu.VMEM((PPW, nch), jnp.float32),
            pltpu.VMEM((PPW, nch), jnp.float32),
            pltpu.SemaphoreType.DMA,
            pltpu.SemaphoreType.DMA,
        ],
    )(functools.partial(_sc_body, nch=nch))


def kernel(x, coords, s, pixel_offset_normal, pixel_offset_indices):
    b, c, nx, ny = x.shape
    _, n, _ = coords.shape
    npts = b * n

    cx = coords[:, :, 1].reshape(NW, PPW)
    cy = coords[:, :, 0].reshape(NW, PPW)
    s1 = jnp.reshape(s, (1,))

    idx, w = _make_prep(nx, ny)(cx, cy, s1, pixel_offset_normal,
                                pixel_offset_indices)

    xt = jnp.transpose(x, (0, 2, 3, 1)).reshape(b * nx * ny, c)
    out = _make_sc(npts, c)(xt, idx, w)
    return out.reshape(b, n, c).transpose(0, 2, 1)
